# baseline (speedup 1.0000x reference)
import functools

import jax
import jax.numpy as jnp
from jax import lax
from jax.experimental import pallas as pl
from jax.experimental.pallas import tpu as pltpu
from jax.experimental.pallas import tpu_sc as plsc

B = 16384
D = 64
WIDTH = 128

_NC = 2
_NS = 16
_NW = _NC * _NS
_BPW = B // _NW
_CHUNK = 128
_NCHUNK = _BPW // _CHUNK


def _sc_gather(e1r, relr, e2r, entity_emb, relation_emb):
    mesh = plsc.VectorSubcoreMesh(core_axis_name="c", subcore_axis_name="s")

    @functools.partial(
        pl.kernel,
        mesh=mesh,
        out_type=(
            jax.ShapeDtypeStruct((B, D), jnp.float32),
            jax.ShapeDtypeStruct((B, D), jnp.float32),
            jax.ShapeDtypeStruct((B, D), jnp.float32),
        ),
        scratch_types=[
            pltpu.VMEM((_NCHUNK, _CHUNK), jnp.int32),
            pltpu.VMEM((_NCHUNK, _CHUNK), jnp.int32),
            pltpu.VMEM((_NCHUNK, _CHUNK), jnp.int32),
            pltpu.VMEM((_BPW, D), jnp.float32),
            pltpu.VMEM((_BPW, D), jnp.float32),
            pltpu.VMEM((_BPW, D), jnp.float32),
            pltpu.SemaphoreType.DMA,
        ],
        compiler_params=pltpu.CompilerParams(use_tc_tiling_on_sc=False),
    )
    def k(e1_hbm, rel_hbm, e2_hbm, ent_hbm, relemb_hbm,
          oh_hbm, or_hbm, ot_hbm,
          ih_v, ir_v, it_v, rh_v, rr_v, rt_v, sem):
        wid = lax.axis_index("s") * _NC + lax.axis_index("c")
        row0 = wid * _NCHUNK
        base = wid * _BPW
        pltpu.sync_copy(e1_hbm.at[pl.ds(row0, _NCHUNK)], ih_v)
        pltpu.sync_copy(rel_hbm.at[pl.ds(row0, _NCHUNK)], ir_v)
        pltpu.sync_copy(e2_hbm.at[pl.ds(row0, _NCHUNK)], it_v)
        copies = []
        for j in range(_NCHUNK):
            dst = pl.ds(j * _CHUNK, _CHUNK)
            copies.append(pltpu.async_copy(
                ent_hbm.at[ih_v.at[j]], rh_v.at[dst], sem))
            copies.append(pltpu.async_copy(
                relemb_hbm.at[ir_v.at[j]], rr_v.at[dst], sem))
            copies.append(pltpu.async_copy(
                ent_hbm.at[it_v.at[j]], rt_v.at[dst], sem))
        for cp in copies:
            cp.wait()
        pltpu.sync_copy(rh_v, oh_hbm.at[pl.ds(base, _BPW)])
        pltpu.sync_copy(rr_v, or_hbm.at[pl.ds(base, _BPW)])
        pltpu.sync_copy(rt_v, ot_hbm.at[pl.ds(base, _BPW)])

    return k(e1r, relr, e2r, entity_emb, relation_emb)


def _mlp_body(h_ref, r_ref, t_ref, w1h, w1r, w1t, b1, gamma, beta, w2, b2,
              out_ref):
    y = jnp.dot(h_ref[...], w1h[...], preferred_element_type=jnp.float32)
    y = y + jnp.dot(r_ref[...], w1r[...], preferred_element_type=jnp.float32)
    y = y + jnp.dot(t_ref[...], w1t[...], preferred_element_type=jnp.float32)
    y = y + b1[...]
    mean = jnp.mean(y, axis=0, keepdims=True)
    yc = y - mean
    var = jnp.mean(yc * yc, axis=0, keepdims=True)
    z = yc * (gamma[...] * lax.rsqrt(var + 1e-5)) + beta[...]
    z = jnp.maximum(z, 0.0)
    o = jnp.dot(z, w2[...], preferred_element_type=jnp.float32) + b2[...]
    out_ref[...] = jax.nn.sigmoid(o)


def kernel(e1_idx, rel_idx, e2_idx, entity_emb, relation_emb,
           W1, b1, gamma, beta, W2, b2):
    e1r = e1_idx.astype(jnp.int32).reshape(B // _CHUNK, _CHUNK)
    relr = rel_idx.astype(jnp.int32).reshape(B // _CHUNK, _CHUNK)
    e2r = e2_idx.astype(jnp.int32).reshape(B // _CHUNK, _CHUNK)

    head, rel, tail = _sc_gather(e1r, relr, e2r, entity_emb, relation_emb)

    out = pl.pallas_call(
        _mlp_body,
        out_shape=jax.ShapeDtypeStruct((B, 1), jnp.float32),
    )(head, rel, tail,
      W1[:D], W1[D:2 * D], W1[2 * D:],
      b1.reshape(1, WIDTH), gamma.reshape(1, WIDTH), beta.reshape(1, WIDTH),
      W2, b2.reshape(1, 1))
    return out
